# baseline (speedup 1.0000x reference)
import jax
import jax.numpy as jnp
from jax import lax
from jax.experimental import pallas as pl
from jax.experimental.pallas import tpu as pltpu


@jax.jit
def _extract_eos_tc(tokens, mask):
    B, N, D = tokens.shape
    HALF = B // 2

    def body(mask_ref, tokens_hbm, out_ref, idx_smem, sem):
        i = pl.program_id(0)

        @pl.when(i == 0)
        def _():
            m = mask_ref[...]
            iota = lax.broadcasted_iota(jnp.int32, (B, N), 1)
            val = jnp.where(m != 0, iota, jnp.int32(N))
            mins = jnp.min(val, axis=1)
            mins = jnp.where(mins >= N, 0, mins)
            for b in range(B):
                idx_smem[b] = mins[b]

        copies = []
        for j in range(HALF):
            b = i * HALF + j
            idx_b = idx_smem[b]
            cp = pltpu.make_async_copy(
                tokens_hbm.at[b, pl.ds(idx_b, 1), :],
                out_ref.at[0, pl.ds(j, 1), :],
                sem,
            )
            cp.start()
            copies.append(cp)
        for cp in copies:
            cp.wait()

    return pl.pallas_call(
        body,
        grid=(2,),
        out_shape=jax.ShapeDtypeStruct((2, HALF, D), tokens.dtype),
        in_specs=[
            pl.BlockSpec((B, N), lambda i: (0, 0)),
            pl.BlockSpec(memory_space=pl.ANY),
        ],
        out_specs=pl.BlockSpec((1, HALF, D), lambda i: (i, 0, 0)),
        scratch_shapes=[
            pltpu.SMEM((B,), jnp.int32),
            pltpu.SemaphoreType.DMA,
        ],
    )(mask, tokens).reshape(B, D)


def kernel(tokens, eos_token_mask):
    return _extract_eos_tc(tokens, eos_token_mask)
